# Initial kernel scaffold; baseline (speedup 1.0000x reference)
import functools

import jax
import jax.numpy as jnp
from jax import lax
from jax.experimental import pallas as pl
from jax.experimental.pallas import tpu as pltpu
from jax.experimental.pallas import tpu_sc as plsc

N = 10000
H = 10000
E = 320000
D = 128

NC = 2
NS = 16
NW = NC * NS
EPW = E // NW
CH = 80
NCHUNK = EPW // CH
RPT = N // NS
ZCH = 125
DEGW = 16

_MESH = plsc.VectorSubcoreMesh(core_axis_name="c", subcore_axis_name="s")


@functools.partial(
    pl.kernel,
    out_type=jax.ShapeDtypeStruct((NC * N, D), jnp.float32),
    mesh=_MESH,
    scratch_types=[
        pltpu.VMEM_SHARED((N, D), jnp.float32),
        pltpu.VMEM((CH,), jnp.int32),
        pltpu.VMEM((CH,), jnp.int32),
        pltpu.VMEM((CH, D), jnp.float32),
        pltpu.VMEM((ZCH, D), jnp.float32),
        pltpu.SemaphoreType.DMA,
    ],
)
def _sc_edge_pass(table, gidx_hbm, sidx_hbm, out, acc, gidx, sidx, rows, zbuf, sem):
    c = lax.axis_index("c")
    s = lax.axis_index("s")
    wid = s * NC + c

    zv = jnp.zeros((16,), jnp.float32)

    @pl.loop(0, ZCH * (D // 16))
    def _zero_zbuf(i):
        r = i // (D // 16)
        col = (i % (D // 16)) * 16
        zbuf[r, pl.ds(col, 16)] = zv

    row0 = s * RPT

    @pl.loop(0, RPT // ZCH)
    def _zero_acc(j):
        pltpu.sync_copy(zbuf, acc.at[pl.ds(row0 + j * ZCH, ZCH)])

    plsc.subcore_barrier()

    base0 = wid * EPW

    @pl.loop(0, NCHUNK)
    def _edges(g):
        b = base0 + g * CH
        pltpu.sync_copy(gidx_hbm.at[pl.ds(b, CH)], gidx)
        pltpu.sync_copy(sidx_hbm.at[pl.ds(b, CH)], sidx)
        pltpu.async_copy(table.at[gidx], rows, sem).wait()
        pltpu.sync_copy(rows, acc.at[sidx], add=True)

    plsc.subcore_barrier()

    @pl.loop(0, RPT // ZCH)
    def _dump(j):
        r = row0 + j * ZCH
        pltpu.sync_copy(acc.at[pl.ds(r, ZCH)], out.at[pl.ds(c * N + r, ZCH)])


@functools.partial(
    pl.kernel,
    out_type=(
        jax.ShapeDtypeStruct((NC * N, DEGW), jnp.float32),
        jax.ShapeDtypeStruct((NC * H, DEGW), jnp.float32),
    ),
    mesh=_MESH,
    scratch_types=[
        pltpu.VMEM_SHARED((N, DEGW), jnp.float32),
        pltpu.VMEM_SHARED((H, DEGW), jnp.float32),
        pltpu.VMEM((CH,), jnp.int32),
        pltpu.VMEM((CH,), jnp.int32),
        pltpu.VMEM((CH, DEGW), jnp.float32),
        pltpu.VMEM((ZCH, DEGW), jnp.float32),
    ],
)
def _sc_degrees(nidx_hbm, hidx_hbm, outn, outh, accn, acch, nidx, hidx, ones, zbuf):
    c = lax.axis_index("c")
    s = lax.axis_index("s")
    wid = s * NC + c

    ov = jnp.ones((16,), jnp.float32)
    zv = jnp.zeros((16,), jnp.float32)

    @pl.loop(0, CH)
    def _fill_ones(i):
        ones[i, pl.ds(0, 16)] = ov

    @pl.loop(0, ZCH)
    def _zero_zbuf(i):
        zbuf[i, pl.ds(0, 16)] = zv

    row0 = s * RPT

    @pl.loop(0, RPT // ZCH)
    def _zero_acc(j):
        pltpu.sync_copy(zbuf, accn.at[pl.ds(row0 + j * ZCH, ZCH)])
        pltpu.sync_copy(zbuf, acch.at[pl.ds(row0 + j * ZCH, ZCH)])

    plsc.subcore_barrier()

    base0 = wid * EPW

    @pl.loop(0, NCHUNK)
    def _edges(g):
        b = base0 + g * CH
        pltpu.sync_copy(nidx_hbm.at[pl.ds(b, CH)], nidx)
        pltpu.sync_copy(hidx_hbm.at[pl.ds(b, CH)], hidx)
        pltpu.sync_copy(ones, accn.at[nidx], add=True)
        pltpu.sync_copy(ones, acch.at[hidx], add=True)

    plsc.subcore_barrier()

    pltpu.sync_copy(accn.at[pl.ds(row0, RPT)], outn.at[pl.ds(c * N + row0, RPT)])
    pltpu.sync_copy(acch.at[pl.ds(row0, RPT)], outh.at[pl.ds(c * H + row0, RPT)])


_BLK = 1250


def _deginv_body(cn_ref, ch_ref, dinv_ref, binv_ref):
    cn = cn_ref[0] + cn_ref[1]
    dinv_ref[...] = jnp.where(cn > 0, 1.0 / cn, 0.0)
    ch = ch_ref[0] + ch_ref[1]
    binv_ref[...] = jnp.where(ch > 0, 1.0 / ch, 0.0)


def _tc_deginv(cn, ch):
    return pl.pallas_call(
        _deginv_body,
        grid=(8,),
        in_specs=[
            pl.BlockSpec((2, _BLK, DEGW), lambda i: (0, i, 0)),
            pl.BlockSpec((2, _BLK, DEGW), lambda i: (0, i, 0)),
        ],
        out_specs=[
            pl.BlockSpec((_BLK, DEGW), lambda i: (i, 0)),
            pl.BlockSpec((_BLK, DEGW), lambda i: (i, 0)),
        ],
        out_shape=[
            jax.ShapeDtypeStruct((N, DEGW), jnp.float32),
            jax.ShapeDtypeStruct((H, DEGW), jnp.float32),
        ],
    )(cn, ch)


def _mm_body(x_ref, w_ref, o_ref):
    o_ref[...] = jnp.dot(x_ref[...], w_ref[...],
                         preferred_element_type=jnp.float32)


def _tc_matmul(x, w):
    return pl.pallas_call(
        _mm_body,
        grid=(8,),
        in_specs=[
            pl.BlockSpec((_BLK, D), lambda i: (i, 0)),
            pl.BlockSpec((D, D), lambda i: (0, 0)),
        ],
        out_specs=pl.BlockSpec((_BLK, D), lambda i: (i, 0)),
        out_shape=jax.ShapeDtypeStruct((N, D), jnp.float32),
    )(x, w)


def _scale_body(inv_ref, p_ref, o_ref):
    o_ref[...] = inv_ref[:, 0:1] * (p_ref[0] + p_ref[1])


def _tc_scale(inv, p):
    return pl.pallas_call(
        _scale_body,
        grid=(8,),
        in_specs=[
            pl.BlockSpec((_BLK, DEGW), lambda i: (i, 0)),
            pl.BlockSpec((2, _BLK, D), lambda i: (0, i, 0)),
        ],
        out_specs=pl.BlockSpec((_BLK, D), lambda i: (i, 0)),
        out_shape=jax.ShapeDtypeStruct((N, D), jnp.float32),
    )(inv, p)


def _layer2_body(dinv_ref, q_ref, b1_ref, w2_ref, o_ref):
    a = dinv_ref[:, 0:1] * (q_ref[0] + q_ref[1]) + b1_ref[...]
    a = jnp.maximum(a, 0.0)
    o_ref[...] = jnp.dot(a, w2_ref[...], preferred_element_type=jnp.float32)


def _tc_layer2(dinv, q, b1, w2):
    return pl.pallas_call(
        _layer2_body,
        grid=(8,),
        in_specs=[
            pl.BlockSpec((_BLK, DEGW), lambda i: (i, 0)),
            pl.BlockSpec((2, _BLK, D), lambda i: (0, i, 0)),
            pl.BlockSpec((1, D), lambda i: (0, 0)),
            pl.BlockSpec((D, D), lambda i: (0, 0)),
        ],
        out_specs=pl.BlockSpec((_BLK, D), lambda i: (i, 0)),
        out_shape=jax.ShapeDtypeStruct((N, D), jnp.float32),
    )(dinv, q, b1, w2)


def _final_body(dinv_ref, q_ref, b2_ref, o_ref):
    o_ref[...] = dinv_ref[:, 0:1] * (q_ref[0] + q_ref[1]) + b2_ref[...]


def _tc_final(dinv, q, b2):
    return pl.pallas_call(
        _final_body,
        grid=(8,),
        in_specs=[
            pl.BlockSpec((_BLK, DEGW), lambda i: (i, 0)),
            pl.BlockSpec((2, _BLK, D), lambda i: (0, i, 0)),
            pl.BlockSpec((1, D), lambda i: (0, 0)),
        ],
        out_specs=pl.BlockSpec((_BLK, D), lambda i: (i, 0)),
        out_shape=jax.ShapeDtypeStruct((N, D), jnp.float32),
    )(dinv, q, b2)


def kernel(x, edge_index, W1, b1, W2, b2):
    nidx = edge_index[0]
    hidx = edge_index[1]
    b1r = b1.reshape(1, D)
    b2r = b2.reshape(1, D)

    cn, chh = _sc_degrees(nidx, hidx)
    dinv, binv = _tc_deginv(cn.reshape(NC, N, DEGW), chh.reshape(NC, H, DEGW))

    h1 = _tc_matmul(x, W1)
    p = _sc_edge_pass(h1, nidx, hidx).reshape(NC, H, D)
    he1 = _tc_scale(binv, p)
    q = _sc_edge_pass(he1, hidx, nidx).reshape(NC, N, D)
    h2 = _tc_layer2(dinv, q, b1r, W2)

    r = _sc_edge_pass(h2, nidx, hidx).reshape(NC, H, D)
    he2 = _tc_scale(binv, r)
    sfin = _sc_edge_pass(he2, hidx, nidx).reshape(NC, N, D)
    return _tc_final(dinv, sfin, b2r)


# speedup vs baseline: 17.2891x; 17.2891x over previous
import functools

import jax
import jax.numpy as jnp
from jax import lax
from jax.experimental import pallas as pl
from jax.experimental.pallas import tpu as pltpu
from jax.experimental.pallas import tpu_sc as plsc

N = 10000
H = 10000
E = 320000
D = 128

NC = 2
NS = 16
NW = NC * NS
EPW = E // NW
CH = 80
PADC = 240
EPWP = EPW + PADC
NCHUNK = EPWP // CH
SB = 8
FB = 2
NH = SB // FB
NSUPER = NCHUNK // SB
NP = 10240
RPT = NP // NS
ZCH = 128
DEGW = 16


@functools.cache
def _sc_edge_pass_kernel():
    mesh = plsc.VectorSubcoreMesh(core_axis_name="c", subcore_axis_name="s",
                                  num_cores=NC, num_subcores=NS)
    return pl.kernel(
        _sc_edge_pass_body,
        out_type=jax.ShapeDtypeStruct((NC * NP, D), jnp.float32),
        mesh=mesh,
        scratch_types=[
            pltpu.VMEM_SHARED((NP, D), jnp.float32),
            pltpu.VMEM((2, SB, CH), jnp.int32),
            pltpu.VMEM((2, SB, CH), jnp.int32),
            pltpu.VMEM((2, FB, CH, D), jnp.float32),
            pltpu.SemaphoreType.DMA,
            pltpu.SemaphoreType.DMA,
            pltpu.SemaphoreType.DMA,
            pltpu.SemaphoreType.DMA,
        ],
    )


def _sc_edge_pass(table, gidx3, sidx3):
    return _sc_edge_pass_kernel()(table, gidx3, sidx3)


def _sc_edge_pass_body(table, gidx_hbm, sidx_hbm, out,
                       acc, gbuf, sbuf, rows, semg, sems0, sems1, semi):
    c = lax.axis_index("c")
    s = lax.axis_index("s")
    wid = s * NC + c

    zv = jnp.zeros((16,), jnp.float32)

    @pl.loop(0, CH * (D // 16))
    def _zero_tile(i):
        r = i // (D // 16)
        col = (i % (D // 16)) * 16
        rows[0, 0, r, pl.ds(col, 16)] = zv

    row0 = s * RPT

    @pl.loop(0, RPT // CH)
    def _zero_acc(j):
        pltpu.sync_copy(rows.at[0, 0], acc.at[pl.ds(row0 + j * CH, CH)])

    plsc.subcore_barrier()

    def _stage(t, slot):
        return [pltpu.async_copy(gidx_hbm.at[wid, pl.ds(t * SB, SB)],
                                 gbuf.at[slot], semi),
                pltpu.async_copy(sidx_hbm.at[wid, pl.ds(t * SB, SB)],
                                 sbuf.at[slot], semi)]

    for dsc in _stage(0, 0):
        dsc.wait()

    def _gather(slot, b):
        return [pltpu.async_copy(table.at[gbuf.at[slot, b * FB + j]],
                                 rows.at[b % 2, j], semg)
                for j in range(FB)]

    def _scatter(slot, b):
        sem = sems0 if b % 2 == 0 else sems1
        return [pltpu.async_copy(rows.at[b % 2, j],
                                 acc.at[sbuf.at[slot, b * FB + j]], sem,
                                 add=True)
                for j in range(FB)]

    @pl.loop(0, NSUPER)
    def _edges(t):
        slot = t % 2
        nstage = _stage((t + 1) % NSUPER, 1 - slot)
        gd = _gather(slot, 0)
        sd_prev = None
        for b in range(NH):
            for dsc in gd:
                dsc.wait()
            sd = _scatter(slot, b)
            if sd_prev is not None:
                for dsc in sd_prev:
                    dsc.wait()
            if b + 1 < NH:
                gd = _gather(slot, b + 1)
            sd_prev = sd
        for dsc in sd_prev:
            dsc.wait()
        for dsc in nstage:
            dsc.wait()

    plsc.subcore_barrier()

    @pl.loop(0, RPT // ZCH)
    def _dump(j):
        r = row0 + j * ZCH
        pltpu.sync_copy(acc.at[pl.ds(r, ZCH)], out.at[pl.ds(c * NP + r, ZCH)])


@functools.cache
def _sc_count_kernel():
    mesh = plsc.VectorSubcoreMesh(core_axis_name="c", subcore_axis_name="s",
                                  num_cores=NC, num_subcores=NS)
    return pl.kernel(
        _sc_count_body,
        out_type=jax.ShapeDtypeStruct((NC * NP, D), jnp.float32),
        mesh=mesh,
        scratch_types=[
            pltpu.VMEM_SHARED((NP, D), jnp.float32),
            pltpu.VMEM((2, SB, CH), jnp.int32),
            pltpu.VMEM((CH, D), jnp.float32),
            pltpu.VMEM((ZCH, D), jnp.float32),
            pltpu.SemaphoreType.DMA,
            pltpu.SemaphoreType.DMA,
        ],
    )


def _sc_count(idx3):
    return _sc_count_kernel()(idx3)


def _sc_count_body(idx_hbm, out, acc, ibuf, ones, zbuf, sems, semi):
    c = lax.axis_index("c")
    s = lax.axis_index("s")
    wid = s * NC + c

    ov = jnp.ones((16,), jnp.float32)
    zv = jnp.zeros((16,), jnp.float32)

    @pl.loop(0, CH * (D // 16))
    def _fill_ones(i):
        r = i // (D // 16)
        col = (i % (D // 16)) * 16
        ones[r, pl.ds(col, 16)] = ov

    @pl.loop(0, ZCH * (D // 16))
    def _zero_zbuf(i):
        r = i // (D // 16)
        col = (i % (D // 16)) * 16
        zbuf[r, pl.ds(col, 16)] = zv

    row0 = s * RPT

    @pl.loop(0, RPT // ZCH)
    def _zero_acc(j):
        pltpu.sync_copy(zbuf, acc.at[pl.ds(row0 + j * ZCH, ZCH)])

    plsc.subcore_barrier()

    def _stage(t, slot):
        return [pltpu.async_copy(idx_hbm.at[wid, pl.ds(t * SB, SB)],
                                 ibuf.at[slot], semi)]

    for dsc in _stage(0, 0):
        dsc.wait()

    @pl.loop(0, NSUPER)
    def _edges(t):
        slot = t % 2
        nstage = _stage((t + 1) % NSUPER, 1 - slot)
        sd = [pltpu.async_copy(ones, acc.at[ibuf.at[slot, j]], sems, add=True)
              for j in range(SB)]
        for dsc in sd:
            dsc.wait()
        for dsc in nstage:
            dsc.wait()

    plsc.subcore_barrier()

    @pl.loop(0, RPT // ZCH)
    def _dump(j):
        r = row0 + j * ZCH
        pltpu.sync_copy(acc.at[pl.ds(r, ZCH)], out.at[pl.ds(c * NP + r, ZCH)])


_BLK = 2000


def _deginv_body(cn_ref, ch_ref, dinv_ref, binv_ref):
    cn = cn_ref[0, :, :DEGW] + cn_ref[1, :, :DEGW]
    dinv_ref[...] = jnp.where(cn > 0, 1.0 / cn, 0.0)
    ch = ch_ref[0, :, :DEGW] + ch_ref[1, :, :DEGW]
    binv_ref[...] = jnp.where(ch > 0, 1.0 / ch, 0.0)


def _tc_deginv(cn, ch):
    return pl.pallas_call(
        _deginv_body,
        grid=(5,),
        in_specs=[
            pl.BlockSpec((2, _BLK, D), lambda i: (0, i, 0)),
            pl.BlockSpec((2, _BLK, D), lambda i: (0, i, 0)),
        ],
        out_specs=[
            pl.BlockSpec((_BLK, DEGW), lambda i: (i, 0)),
            pl.BlockSpec((_BLK, DEGW), lambda i: (i, 0)),
        ],
        out_shape=[
            jax.ShapeDtypeStruct((N, DEGW), jnp.float32),
            jax.ShapeDtypeStruct((H, DEGW), jnp.float32),
        ],
    )(cn, ch)


def _mm_body(x_ref, w_ref, o_ref):
    o_ref[...] = jnp.dot(x_ref[...], w_ref[...],
                         preferred_element_type=jnp.float32)


def _tc_matmul(x, w):
    return pl.pallas_call(
        _mm_body,
        grid=(5,),
        in_specs=[
            pl.BlockSpec((_BLK, D), lambda i: (i, 0)),
            pl.BlockSpec((D, D), lambda i: (0, 0)),
        ],
        out_specs=pl.BlockSpec((_BLK, D), lambda i: (i, 0)),
        out_shape=jax.ShapeDtypeStruct((N, D), jnp.float32),
    )(x, w)


def _scale_body(inv_ref, p_ref, o_ref):
    o_ref[...] = inv_ref[:, 0:1] * (p_ref[0] + p_ref[1])


def _tc_scale(inv, p):
    return pl.pallas_call(
        _scale_body,
        grid=(5,),
        in_specs=[
            pl.BlockSpec((_BLK, DEGW), lambda i: (i, 0)),
            pl.BlockSpec((2, _BLK, D), lambda i: (0, i, 0)),
        ],
        out_specs=pl.BlockSpec((_BLK, D), lambda i: (i, 0)),
        out_shape=jax.ShapeDtypeStruct((N, D), jnp.float32),
    )(inv, p)


def _layer2_body(dinv_ref, q_ref, b1_ref, w2_ref, o_ref):
    a = dinv_ref[:, 0:1] * (q_ref[0] + q_ref[1]) + b1_ref[...]
    a = jnp.maximum(a, 0.0)
    o_ref[...] = jnp.dot(a, w2_ref[...], preferred_element_type=jnp.float32)


def _tc_layer2(dinv, q, b1, w2):
    return pl.pallas_call(
        _layer2_body,
        grid=(5,),
        in_specs=[
            pl.BlockSpec((_BLK, DEGW), lambda i: (i, 0)),
            pl.BlockSpec((2, _BLK, D), lambda i: (0, i, 0)),
            pl.BlockSpec((1, D), lambda i: (0, 0)),
            pl.BlockSpec((D, D), lambda i: (0, 0)),
        ],
        out_specs=pl.BlockSpec((_BLK, D), lambda i: (i, 0)),
        out_shape=jax.ShapeDtypeStruct((N, D), jnp.float32),
    )(dinv, q, b1, w2)


def _final_body(dinv_ref, q_ref, b2_ref, o_ref):
    o_ref[...] = dinv_ref[:, 0:1] * (q_ref[0] + q_ref[1]) + b2_ref[...]


def _tc_final(dinv, q, b2):
    return pl.pallas_call(
        _final_body,
        grid=(5,),
        in_specs=[
            pl.BlockSpec((_BLK, DEGW), lambda i: (i, 0)),
            pl.BlockSpec((2, _BLK, D), lambda i: (0, i, 0)),
            pl.BlockSpec((1, D), lambda i: (0, 0)),
        ],
        out_specs=pl.BlockSpec((_BLK, D), lambda i: (i, 0)),
        out_shape=jax.ShapeDtypeStruct((N, D), jnp.float32),
    )(dinv, q, b2)


def kernel(x, edge_index, W1, b1, W2, b2):
    nidx = edge_index[0]
    hidx = edge_index[1]
    b1r = b1.reshape(1, D)
    b2r = b2.reshape(1, D)

    nidx2 = nidx.reshape(NW, EPW)
    hidx2 = hidx.reshape(NW, EPW)
    w = jnp.arange(NW, dtype=jnp.int32)[:, None]
    j = jnp.arange(PADC, dtype=jnp.int32)[None, :]
    padg = (w * PADC + j) % N
    pads = jnp.broadcast_to(N + j, (NW, PADC))
    nidx_g = jnp.concatenate([nidx2, padg], 1).reshape(NW, NCHUNK, CH)
    nidx_s = jnp.concatenate([nidx2, pads], 1).reshape(NW, NCHUNK, CH)
    hidx_g = jnp.concatenate([hidx2, padg], 1).reshape(NW, NCHUNK, CH)
    hidx_s = jnp.concatenate([hidx2, pads], 1).reshape(NW, NCHUNK, CH)

    cn = _sc_count(nidx_s).reshape(NC, NP, D)
    chh = _sc_count(hidx_s).reshape(NC, NP, D)
    dinv, binv = _tc_deginv(cn, chh)

    h1 = _tc_matmul(x, W1)
    p = _sc_edge_pass(h1, nidx_g, hidx_s).reshape(NC, NP, D)
    he1 = _tc_scale(binv, p)
    q = _sc_edge_pass(he1, hidx_g, nidx_s).reshape(NC, NP, D)
    h2 = _tc_layer2(dinv, q, b1r, W2)

    r = _sc_edge_pass(h2, nidx_g, hidx_s).reshape(NC, NP, D)
    he2 = _tc_scale(binv, r)
    sfin = _sc_edge_pass(he2, hidx_g, nidx_s).reshape(NC, NP, D)
    return _tc_final(dinv, sfin, b2r)
